# Initial kernel scaffold; baseline (speedup 1.0000x reference)
import jax
import jax.numpy as jnp
from jax.experimental import pallas as pl


def kernel(H, edge_weights):
    raise NotImplementedError("write your pallas kernel here")


# speedup vs baseline: 15.9972x; 15.9972x over previous
import functools

import jax
import jax.numpy as jnp
from jax import lax
from jax.experimental import pallas as pl
from jax.experimental.pallas import tpu as pltpu
from jax.experimental.pallas import tpu_sc as plsc

_N_NODES = 10000
_N_EDGES = 320000
_H = 128
_BATCH = 2

_NC = 2
_NS = 16
_L = 16

_EC = 80
_EDGES_PER_TILE = _N_EDGES // _NS
_CHUNKS = _EDGES_PER_TILE // _EC
_WB = 40
_WB_TOTAL = _N_NODES // _WB
_WB_PER_TILE = (_WB_TOTAL + _NS - 1) // _NS

_mesh = plsc.VectorSubcoreMesh(core_axis_name="c", subcore_axis_name="s")


@functools.partial(
    pl.kernel,
    out_type=jax.ShapeDtypeStruct((_BATCH, _N_NODES, _H), jnp.float32),
    mesh=_mesh,
    scratch_types=[
        pltpu.VMEM_SHARED((_N_NODES, _H), jnp.float32),
        pltpu.VMEM((_EC,), jnp.int32),
        pltpu.VMEM((_EC,), jnp.int32),
        pltpu.VMEM((_EC,), jnp.float32),
        pltpu.VMEM((_EC, _H), jnp.float32),
        pltpu.VMEM((_WB, _H), jnp.float32),
    ],
)
def _neighbor_agg(h_ref, n1_ref, n2_ref, w_ref, out_ref,
                  acc, idx1, idx2, wv, rows, zb):
    c = lax.axis_index("c")
    s = lax.axis_index("s")

    zero = jnp.zeros((_L,), jnp.float32)

    def zrow(r, carry):
        for f in range(_H // _L):
            zb[r, pl.ds(f * _L, _L)] = zero
        return carry

    lax.fori_loop(0, _WB, zrow, 0)
    for k in range(_WB_PER_TILE):
        m = s + _NS * k

        @pl.when(m < _WB_TOTAL)
        def _():
            pltpu.sync_copy(zb, acc.at[pl.ds(m * _WB, _WB)])

    plsc.subcore_barrier()

    base0 = c * _N_EDGES + s * _EDGES_PER_TILE

    def chunk(i, carry):
        base = base0 + i * _EC
        pltpu.sync_copy(n1_ref.at[pl.ds(base, _EC)], idx1)
        pltpu.sync_copy(n2_ref.at[pl.ds(base, _EC)], idx2)
        pltpu.sync_copy(w_ref.at[pl.ds(base, _EC)], wv)
        pltpu.sync_copy(h_ref.at[idx2], rows)
        for g in range(_EC // _L):
            w16 = wv[pl.ds(g * _L, _L)]
            for j in range(_L):
                e = g * _L + j
                ws = w16[j]
                for f in range(_H // _L):
                    sl = pl.ds(f * _L, _L)
                    rows[e, sl] = rows[e, sl] * ws
        pltpu.sync_copy(rows, acc.at[idx1], add=True)
        return carry

    lax.fori_loop(0, _CHUNKS, chunk, 0)
    plsc.subcore_barrier()

    for k in range(_WB_PER_TILE):
        m = s + _NS * k

        @pl.when(m < _WB_TOTAL)
        def _():
            pltpu.sync_copy(acc.at[pl.ds(m * _WB, _WB)], zb)
            pltpu.sync_copy(zb, out_ref.at[c, pl.ds(m * _WB, _WB)])


def kernel(H, edge_weights):
    n1 = edge_weights[..., 0].astype(jnp.int32)
    n2 = edge_weights[..., 1].astype(jnp.int32)
    w = edge_weights[..., 2]
    offs = (jnp.arange(_BATCH, dtype=jnp.int32) * _N_NODES)[:, None]
    n2g = (n2 + offs).reshape(-1)
    h_flat = H.reshape(_BATCH * _N_NODES, _H)
    return _neighbor_agg(h_flat, n1.reshape(-1), n2g, w.reshape(-1))
